# baseline (speedup 1.0000x reference)
import functools
import math

import jax
import jax.numpy as jnp
from jax import lax
from jax.experimental import pallas as pl
from jax.experimental.pallas import tpu as pltpu
from jax.experimental.pallas import tpu_sc as plsc

_LANE = 16
_TL = 128


def _mesh():
    return plsc.VectorSubcoreMesh(core_axis_name="c", subcore_axis_name="s")


def _wid(nc):
    return lax.axis_index("s") * nc + lax.axis_index("c")


@functools.lru_cache(maxsize=None)
def _make_retile(V, D, S):
    info = plsc.get_sparse_core_info()
    nc, ns = info.num_cores, info.num_subcores
    nw = nc * ns
    n_blk = V // _TL
    rem = V - n_blk * _TL
    n_outer = (n_blk + nw - 1) // nw
    rows_per_blk = _TL * D // _TL
    assert D % 8 == 0 and (rem * D) % _TL == 0

    @functools.partial(
        pl.kernel,
        mesh=_mesh(),
        compiler_params=pltpu.CompilerParams(use_tc_tiling_on_sc=True, needs_layout_passes=False),
        out_type=jax.ShapeDtypeStruct((V * D // _TL, _TL), jnp.float32),
        scratch_types=[
            pltpu.VMEM((S, D, _TL), jnp.float32),
            pltpu.VMEM((S, rows_per_blk, _TL), jnp.float32),
            pltpu.VMEM((D, rem if rem else 1), jnp.float32),
            pltpu.VMEM((max(rem * D // _TL, 1), _TL), jnp.float32),
        ]
        + [pltpu.SemaphoreType.DMA] * (2 * S),
    )
    def retile(tt_hbm, out_hbm, tin_v, tout_v, rin_v, rout_v, *sems):
        sem_g = sems[:S]
        sem_w = sems[S:]
        w = _wid(nc)

        def fire_read(b, c):
            pltpu.async_copy(
                tt_hbm.at[:, pl.ds(c * _TL, _TL)], tin_v.at[b], sem_g[b]
            )

        def drain_read(b):
            pltpu.make_async_copy(
                tt_hbm.at[:, pl.ds(0, _TL)], tin_v.at[b], sem_g[b]
            ).wait()

        def wait_write(b):
            pltpu.make_async_copy(
                tout_v.at[b], out_hbm.at[pl.ds(0, rows_per_blk)], sem_w[b]
            ).wait()

        def transpose_blk(b):
            def vrow(v, carry):
                for h in range(0, D, _LANE):
                    col = jnp.full((_LANE,), 0, jnp.int32) + v
                    row = lax.iota(jnp.int32, _LANE) + h
                    vals = plsc.load_gather(tin_v.at[b], [row, col])
                    flat = v * D + h
                    tout_v[b, flat // _TL, pl.ds(flat % _TL, _LANE)] = vals
                return carry

            lax.fori_loop(0, _TL, vrow, 0, unroll=4)

        for b in range(S):
            c = b * nw + w

            @pl.when(c < n_blk)
            def _():
                fire_read(b, c)

        def outer(g, carry):
            for b in range(S):
                c = (g * S + b) * nw + w

                @pl.when(c < n_blk)
                def _():
                    drain_read(b)
                    transpose_blk(b)
                    pltpu.async_copy(
                        tout_v.at[b],
                        out_hbm.at[pl.ds(c * rows_per_blk, rows_per_blk)],
                        sem_w[b],
                    )
                    t = c + S * nw
                    @pl.when(t < n_blk)
                    def _():
                        wait_write(b)
                        fire_read(b, t)
            return carry

        lax.fori_loop(0, n_outer // S + 1, outer, 0)
        for b in range(S):
            c0 = b * nw + w

            @pl.when(c0 < n_blk)
            def _():
                wait_write(b)

        if rem:
            @pl.when(w == nw - 1)
            def _():
                pltpu.sync_copy(tt_hbm.at[:, pl.ds(n_blk * _TL, rem)], rin_v)

                def vrow(v, carry):
                    for h in range(0, D, _LANE):
                        col = jnp.full((_LANE,), 0, jnp.int32) + v
                        row = lax.iota(jnp.int32, _LANE) + h
                        vals = plsc.load_gather(rin_v, [row, col])
                        flat = v * D + h
                        rout_v[flat // _TL, pl.ds(flat % _TL, _LANE)] = vals
                    return carry

                lax.fori_loop(0, rem, vrow, 0, unroll=4)
                pltpu.sync_copy(
                    rout_v,
                    out_hbm.at[pl.ds(n_blk * rows_per_blk, rem * D // _TL)],
                )

    return retile


@functools.lru_cache(maxsize=None)
def _make_gather(BT, F, V, D, S):
    info = plsc.get_sparse_core_info()
    nc, ns = info.num_cores, info.num_subcores
    nw = nc * ns
    n_chunk = F * (BT // _TL)
    per_w = n_chunk // nw
    bblk = BT // _TL
    scale = math.sqrt(D)
    assert BT % _TL == 0 and n_chunk % nw == 0 and per_w % S == 0

    @functools.partial(
        pl.kernel,
        mesh=_mesh(),
        compiler_params=pltpu.CompilerParams(use_tc_tiling_on_sc=True, needs_layout_passes=False),
        out_type=jax.ShapeDtypeStruct((F, D, BT), jnp.float32),
        scratch_types=[
            pltpu.VMEM((S, _TL), jnp.int32),
            pltpu.VMEM((S, _TL), jnp.int32),
            pltpu.VMEM((S, _TL), jnp.int32),
            pltpu.VMEM((S, _TL, _TL), jnp.float32),
            pltpu.VMEM((S, D, _TL), jnp.float32),
        ]
        + [pltpu.SemaphoreType.DMA] * (2 * S),
    )
    def gatherk(xt_hbm, tab_hbm, out_hbm, ixr_v, ix4_v, q_v, rows_v,
                outt_v, *sems):
        sem_g = sems[:S]
        sem_w = sems[S:]
        w = _wid(nc)
        cbase = w * per_w

        def fetch_and_fire(b, c):
            f = c // bblk
            b0 = (c % bblk) * _TL
            pltpu.sync_copy(xt_hbm.at[f, pl.ds(b0, _TL)], ixr_v.at[b])

            def prep(i, carry):
                raw = ixr_v[b, pl.ds(i * _LANE, _LANE)]
                ix4_v[b, pl.ds(i * _LANE, _LANE)] = raw >> 2
                q_v[b, pl.ds(i * _LANE, _LANE)] = raw & 3
                return carry

            lax.fori_loop(0, _TL // _LANE, prep, 0, unroll=8)
            pltpu.async_copy(tab_hbm.at[ix4_v.at[b]], rows_v.at[b], sem_g[b])

        def drain_gather(b):
            pltpu.make_async_copy(
                tab_hbm.at[pl.ds(0, _TL)], rows_v.at[b], sem_g[b]
            ).wait()

        def wait_write(b):
            pltpu.make_async_copy(
                outt_v.at[b], out_hbm.at[0, :, pl.ds(0, _TL)], sem_w[b]
            ).wait()

        def extract(b):
            def bg_body(i, carry):
                lanes = lax.iota(jnp.int32, _LANE) + i * _LANE
                qx = q_v[b, pl.ds(i * _LANE, _LANE)] * D
                for d in range(D):
                    vals = plsc.load_gather(rows_v.at[b], [lanes, qx + d])
                    outt_v[b, d, pl.ds(i * _LANE, _LANE)] = vals * scale
                return carry

            lax.fori_loop(0, _TL // _LANE, bg_body, 0)

        for b in range(S):
            fetch_and_fire(b, cbase + b)

        def outer(g, carry):
            for b in range(S):
                c = cbase + g * S + b
                drain_gather(b)
                extract(b)
                f = c // bblk
                b0 = (c % bblk) * _TL
                pltpu.async_copy(
                    outt_v.at[b],
                    out_hbm.at[f, :, pl.ds(b0, _TL)],
                    sem_w[b],
                )
                bp = (b - 1) % S
                t = c + S - 1
                ok = jnp.logical_and(g * S + b >= 1,
                                     t <= cbase + per_w - 1)

                @pl.when(ok)
                def _():
                    wait_write(bp)
                    fetch_and_fire(bp, t)
            return carry

        lax.fori_loop(0, per_w // S, outer, 0)
        for b in range(S):
            wait_write(b)

    return gatherk


def kernel(x, table):
    bt, f = x.shape
    v, d = table.shape
    tab_rm = _make_retile(v, d, 3)(table.T)
    out_t = _make_gather(bt, f, v, d, 4)(x.T.astype(jnp.int32), tab_rm)
    return out_t.transpose(2, 0, 1)
